# baseline (speedup 1.0000x reference)
import functools

import jax
import jax.numpy as jnp
from jax import lax
from jax.experimental import pallas as pl
from jax.experimental.pallas import tpu as pltpu
from jax.experimental.pallas import tpu_sc as plsc

N = 10000
E = 160000
D = 256
HID = 128

NC = 2
NS = 16
EPW = 5120
EPAD = EPW * NC * NS
CHUNK = 128
NACC = 10240


def _sc_edge_probe_body(h_hbm, src_hbm, dst_hbm, asrc_hbm,
                        num_out, den_out,
                        idxv, dstv, rows, exv, asrc_v, denom_v, acc, sem):
    cid = lax.axis_index("c")
    sid = lax.axis_index("s")
    wid = sid * NC + cid
    h2 = h_hbm

    pltpu.sync_copy(asrc_hbm, asrc_v)

    def _z(i, _):
        r = i // 8
        c = (i % 8) * 16
        rows[r, pl.ds(c, 16)] = jnp.zeros((16,), jnp.float32)
        return 0
    lax.fori_loop(0, CHUNK * 8, _z, 0)

    def _zd(i, _):
        denom_v[pl.ds(i * 16, 16)] = jnp.zeros((16,), jnp.float32)
        return 0
    lax.fori_loop(0, N // 16, _zd, 0)

    for k in range(NACC // NS // CHUNK):
        pltpu.sync_copy(rows, acc.at[pl.ds(sid * (NACC // NS) + k * CHUNK,
                                           CHUNK), :])
    plsc.subcore_barrier()

    def _chunk(c, _):
        off = wid * EPW + c * CHUNK
        pltpu.sync_copy(src_hbm.at[pl.ds(off, CHUNK)], idxv)
        pltpu.sync_copy(dst_hbm.at[pl.ds(off, CHUNK)], dstv)
        pltpu.async_copy(h2.at[idxv], rows, sem).wait()

        def _ex(j, _):
            s16 = idxv[pl.ds(j * 16, 16)]
            d16 = dstv[pl.ds(j * 16, 16)]
            a = plsc.load_gather(asrc_v, [s16]) + plsc.load_gather(asrc_v,
                                                                   [d16])
            a = jnp.where(a >= 0, a, 0.2 * a)
            e = jnp.exp(a - 5.0)
            eid = off + j * 16 + lax.iota(jnp.int32, 16)
            e = jnp.where(eid < E, e, 0.0)
            exv[pl.ds(j * 16, 16)] = e
            plsc.addupdate_scatter(denom_v, [d16], e)
            return 0
        lax.fori_loop(0, CHUNK // 16, _ex, 0)

        def _scale(t, _):
            f = plsc.load_gather(exv, [jnp.full((16,), t, jnp.int32)])
            for j in range(HID // 16):
                rows[t, pl.ds(j * 16, 16)] = rows[t, pl.ds(j * 16, 16)] * f
            return 0
        lax.fori_loop(0, CHUNK, _scale, 0)

        pltpu.async_copy(rows, acc.at[dstv], sem, add=True).wait()
        return 0

    lax.fori_loop(0, EPW // CHUNK, _chunk, 0)
    plsc.subcore_barrier()

    num3 = num_out
    rows_per_sub = NACC // NS
    pltpu.sync_copy(acc.at[pl.ds(sid * rows_per_sub, rows_per_sub), :],
                    num3.at[cid, pl.ds(sid * rows_per_sub, rows_per_sub), :])
    pltpu.sync_copy(denom_v, den_out.at[pl.ds(wid * N, N)])


def _sc_edge_probe(h, src, dst, asrc):
    mesh = plsc.VectorSubcoreMesh(core_axis_name="c", subcore_axis_name="s")
    f = pl.kernel(
        _sc_edge_probe_body,
        out_type=(
            jax.ShapeDtypeStruct((NC, NACC, HID), jnp.float32),
            jax.ShapeDtypeStruct((NC * NS * N,), jnp.float32),
        ),
        mesh=mesh,
        compiler_params=pltpu.CompilerParams(needs_layout_passes=False),
        scratch_types=[
            pltpu.VMEM((CHUNK,), jnp.int32),
            pltpu.VMEM((CHUNK,), jnp.int32),
            pltpu.VMEM((CHUNK, HID), jnp.float32),
            pltpu.VMEM((CHUNK,), jnp.float32),
            pltpu.VMEM((N,), jnp.float32),
            pltpu.VMEM((N,), jnp.float32),
            pltpu.VMEM_SHARED((NACC, HID), jnp.float32),
            pltpu.SemaphoreType.DMA,
        ],
    )
    return f(h, src, dst, asrc)


def _gat(x, edge_index, W, att_src, att_dst, bias, heads, out_ch, concat):
    n = x.shape[0]
    h = jnp.dot(x, W).reshape(n, heads, out_ch)
    src = edge_index[0]
    dst = edge_index[1]
    a_src = jnp.sum(h * att_src, axis=-1)
    a_dst = jnp.sum(h * att_dst, axis=-1)
    alpha = jax.nn.leaky_relu(a_src[src] + a_dst[dst], 0.2)
    amax = jax.ops.segment_max(alpha, dst, num_segments=n)
    amax = jnp.where(jnp.isfinite(amax), amax, 0.0)
    ex = jnp.exp(alpha - amax[dst])
    denom = jax.ops.segment_sum(ex, dst, num_segments=n)
    coef = ex / (denom[dst] + 1e-16)
    out = jax.ops.segment_sum(h[src] * coef[:, :, None], dst, num_segments=n)
    if concat:
        out = out.reshape(n, heads * out_ch)
    else:
        out = jnp.mean(out, axis=1)
    return out + bias


def kernel(node_features, edge_index, question_emb, W1, att_src1, att_dst1,
           b1, W2, att_src2, att_dst2, b2, Wsa, bsa, Wsb, bsb, Wma, bma, Wmb,
           bmb, Wla, bla, Wlb, blb, Wq1, bq1, Wq2, bq2):
    src = edge_index[0]
    dst = edge_index[1]
    npad = EPAD - E
    src_p = jnp.concatenate([src, jnp.zeros((npad,), jnp.int32)])
    dst_p = jnp.concatenate([dst, jnp.zeros((npad,), jnp.int32)])
    h_tab = node_features[:, :HID]
    asrc_t = node_features[:, 200]
    num_o, den_o = _sc_edge_probe(h_tab, src_p, dst_p, asrc_t)
    num_k = num_o[:, :N].sum(0)
    den_k = den_o.reshape(NC * NS, N).sum(0)

    al = jax.nn.leaky_relu(asrc_t[src] + asrc_t[dst], 0.2)
    ex = jnp.exp(al - 5.0)
    num_e = jax.ops.segment_sum(h_tab[src] * ex[:, None], dst,
                                num_segments=N)
    den_e = jax.ops.segment_sum(ex, dst, num_segments=N)
    delta = jnp.mean(jnp.abs(num_k - num_e)) + jnp.mean(jnp.abs(den_k - den_e))

    x = jax.nn.relu(_gat(node_features, edge_index, W1, att_src1, att_dst1,
                         b1, 4, HID, True))
    x = jax.nn.relu(_gat(x, edge_index, W2, att_src2, att_dst2, b2, 1, HID,
                         False))
    g = jnp.mean(x, axis=0, keepdims=True)
    structural = jax.nn.relu(g @ Wsa + bsa) @ Wsb + bsb
    semantic = jax.nn.relu(g @ Wma + bma) @ Wmb + bmb
    logic = jax.nn.relu(g @ Wla + bla) @ Wlb + blb
    q = jax.nn.relu(question_emb @ Wq1 + bq1) @ Wq2 + bq2
    q = q[None, :]
    attn = jax.nn.softmax(g @ q.T, axis=-1)
    semantic = semantic * attn
    total = structural + semantic + logic + delta
    return (structural, semantic, logic, total)


# speedup vs baseline: 4.8760x; 4.8760x over previous
import jax
import jax.numpy as jnp
from jax import lax
from jax.experimental import pallas as pl
from jax.experimental.pallas import tpu as pltpu
from jax.experimental.pallas import tpu_sc as plsc

N = 10000
E = 160000
D = 256
HID = 128

NC = 2
NS = 16
NW = NC * NS
NP = 10240
NPW = NP // NW
NPS = NP // NS
EPW = 5120
EPAD = EPW * NW
CHUNK = 80
BL = 128 * 128

_F32 = jnp.float32
_I32 = jnp.int32


def _mesh():
    return plsc.VectorSubcoreMesh(core_axis_name="c", subcore_axis_name="s")


def _params():
    return pltpu.CompilerParams(needs_layout_passes=False)


def _splat(ref, idx):
    return plsc.load_gather(ref, [jnp.full((16,), idx, _I32)])


def _zeros16():
    return jnp.zeros((16,), _F32)


def _lrelu(v):
    return jnp.where(v >= 0, v, 0.2 * v)


def _make_mm_body(K, JC, KB, jgroups):
    nin = 1 + KB * JC

    def body(*refs):
        x_hbm = refs[0]
        wrefs = refs[1:nin]
        outs = refs[nin:nin + JC]
        xbuf, wstage, houtA, houtB = refs[nin + JC:nin + JC + 4]
        houts = [houtA, houtB]
        cid = lax.axis_index("c")
        sid = lax.axis_index("s")
        wid = sid * NC + cid

        for gj in jgroups:
            for jj, j in enumerate(gj):
                for kb in range(KB):
                    pltpu.sync_copy(
                        wrefs[kb * JC + j],
                        wstage.at[pl.ds((jj * KB + kb) * BL, BL)])

            def _chunk(c, _):
                nbase = wid * NPW + c * 32
                pltpu.sync_copy(x_hbm.at[pl.ds(nbase * K, 32 * K)], xbuf)

                def _node(n, _):
                    for jj, j in enumerate(gj):
                        accs = tuple(_zeros16() for _ in range(8))
                        for kb in range(KB):
                            woff = (jj * KB + kb) * BL

                            def _k(k, a, kb=kb, woff=woff):
                                xs = _splat(xbuf, n * K + kb * 128 + k)
                                return tuple(
                                    a[r] + xs * wstage[pl.ds(
                                        woff + k * 128 + r * 16, 16)]
                                    for r in range(8))
                            accs = lax.fori_loop(0, 128, _k, accs)
                        for r in range(8):
                            houts[jj][n, pl.ds(r * 16, 16)] = accs[r]
                    return 0
                lax.fori_loop(0, 32, _node, 0)
                for jj, j in enumerate(gj):
                    pltpu.sync_copy(houts[jj],
                                    outs[j].at[pl.ds(nbase, 32), :])
                return 0
            lax.fori_loop(0, NPW // 32, _chunk, 0)

    return body


def _mm(x_flat, wblocks, K, JC, KB, jgroups):
    f = pl.kernel(
        _make_mm_body(K, JC, KB, jgroups),
        out_type=tuple(jax.ShapeDtypeStruct((NP, 128), _F32)
                       for _ in range(JC)),
        mesh=_mesh(),
        compiler_params=_params(),
        scratch_types=[
            pltpu.VMEM((32 * K,), _F32),
            pltpu.VMEM((2 * 2 * BL,), _F32),
            pltpu.VMEM((32, 128), _F32),
            pltpu.VMEM((32, 128), _F32),
        ],
    )
    outs = f(x_flat, *wblocks)
    if JC == 1 and not isinstance(outs, (tuple, list)):
        outs = (outs,)
    return outs


def _make_edge_body(H):
    def body(*refs):
        hrefs = refs[0:H]
        atts_h, attd_h, src_h, dst_h = refs[H:H + 4]
        nums = refs[H + 4:H + 4 + H]
        dens = refs[H + 4 + H:H + 4 + 2 * H]
        atab, dgrid, mxg = refs[H + 4 + 2 * H:H + 7 + 2 * H]
        (idxv, dstv, rows, exv, asrc_v, adst_v, denom_v, hbuf,
         attsv, attdv, aslocal, adlocal, dgr, denb, mgv, maxb,
         acc, sem) = refs[H + 7 + 2 * H:]
        cid = lax.axis_index("c")
        sid = lax.axis_index("s")
        wid = sid * NC + cid

        pltpu.sync_copy(atts_h, attsv)
        pltpu.sync_copy(attd_h, attdv)

        pmax = jnp.full((16,), -1e30, _F32)
        for h in range(H):
            def _pch(c, carry, h=h):
                mxs, mxd = carry
                n0 = sid * NPS + c * 16
                pltpu.sync_copy(hrefs[h].at[pl.ds(n0, 16), :], hbuf)
                accs = _zeros16()
                accd = _zeros16()

                def _cc(ccol, car, h=h):
                    a_s, a_d = car
                    ridx = lax.iota(_I32, 16)
                    hv = plsc.load_gather(
                        hbuf, [ridx, jnp.full((16,), ccol, _I32)])
                    ws = _splat(attsv, h * 128 + ccol)
                    wd = _splat(attdv, h * 128 + ccol)
                    return (a_s + hv * ws, a_d + hv * wd)
                accs, accd = lax.fori_loop(0, 128, _cc, (accs, accd))
                aslocal[pl.ds(c * 16, 16)] = accs
                adlocal[pl.ds(c * 16, 16)] = accd
                mxs = jnp.maximum(mxs, accs)
                mxd = jnp.maximum(mxd, accd)
                return (mxs, mxd)
            mxs, mxd = lax.fori_loop(
                0, NPS // 16, _pch,
                (jnp.full((16,), -1e30, _F32), jnp.full((16,), -1e30, _F32)))
            pltpu.sync_copy(aslocal, atab.at[h, 0, pl.ds(sid * NPS, NPS)])
            pltpu.sync_copy(adlocal, atab.at[h, 1, pl.ds(sid * NPS, NPS)])
            ms = jnp.max(mxs)
            md = jnp.max(mxd)
            lane = lax.iota(_I32, 16)
            pmax = jnp.where(lane == 2 * h, jnp.full((16,), ms), pmax)
            pmax = jnp.where(lane == 2 * h + 1, jnp.full((16,), md), pmax)
        maxb[pl.ds(0, 16)] = pmax
        pltpu.sync_copy(maxb, mxg.at[cid, sid])
        plsc.subcore_barrier()

        pltpu.sync_copy(mxg.at[cid], mgv)
        macc = mgv[0]
        for r in range(1, NS):
            macc = jnp.maximum(macc, mgv[r])
        maxb[pl.ds(0, 16)] = macc

        for h in range(H):
            pltpu.sync_copy(atab.at[h, 0], asrc_v)
            pltpu.sync_copy(atab.at[h, 1], adst_v)
            gv = _lrelu(_splat(maxb, 2 * h) + _splat(maxb, 2 * h + 1))

            def _zd(i, _):
                denom_v[pl.ds(i * 16, 16)] = _zeros16()
                return 0
            lax.fori_loop(0, NP // 16, _zd, 0)

            def _zr(i, _):
                rr = i // 8
                cc = (i % 8) * 16
                rows[rr, pl.ds(cc, 16)] = _zeros16()
                return 0
            lax.fori_loop(0, CHUNK * 8, _zr, 0)
            for k in range(NPS // CHUNK):
                pltpu.sync_copy(
                    rows, acc.at[pl.ds(sid * NPS + k * CHUNK, CHUNK), :])
            plsc.subcore_barrier()

            def _chunk(c, _, h=h, gv=gv):
                off = wid * EPW + c * CHUNK
                pltpu.sync_copy(src_h.at[pl.ds(off, CHUNK)], idxv)
                pltpu.sync_copy(dst_h.at[pl.ds(off, CHUNK)], dstv)
                pltpu.async_copy(hrefs[h].at[idxv], rows, sem).wait()

                def _ex(j, _):
                    s16 = idxv[pl.ds(j * 16, 16)]
                    d16 = dstv[pl.ds(j * 16, 16)]
                    a = (plsc.load_gather(asrc_v, [s16])
                         + plsc.load_gather(adst_v, [d16]))
                    e = jnp.exp(_lrelu(a) - gv)
                    eid = off + j * 16 + lax.iota(_I32, 16)
                    e = jnp.where(eid < E, e, 0.0)
                    exv[pl.ds(j * 16, 16)] = e
                    plsc.addupdate_scatter(denom_v, [d16], e)
                    return 0
                lax.fori_loop(0, CHUNK // 16, _ex, 0)

                def _scale(t, _):
                    f = _splat(exv, t)
                    for r in range(8):
                        rows[t, pl.ds(r * 16, 16)] = (
                            rows[t, pl.ds(r * 16, 16)] * f)
                    return 0
                lax.fori_loop(0, CHUNK, _scale, 0)

                pltpu.async_copy(rows, acc.at[dstv], sem, add=True).wait()
                return 0
            lax.fori_loop(0, EPW // CHUNK, _chunk, 0)
            plsc.subcore_barrier()

            pltpu.sync_copy(
                acc.at[pl.ds(sid * NPS, NPS), :],
                nums[h].at[cid, pl.ds(sid * NPS, NPS), :])
            pltpu.sync_copy(denom_v, dgrid.at[cid, sid])
            plsc.subcore_barrier()

            def _zb(i, _):
                denb[pl.ds(i * 16, 16)] = _zeros16()
                return 0
            lax.fori_loop(0, NPS // 16, _zb, 0)
            for r in range(NS):
                pltpu.sync_copy(dgrid.at[cid, r, pl.ds(sid * NPS, NPS)],
                                dgr)

                def _dred(i, _):
                    denb[pl.ds(i * 16, 16)] = (
                        denb[pl.ds(i * 16, 16)] + dgr[pl.ds(i * 16, 16)])
                    return 0
                lax.fori_loop(0, NPS // 16, _dred, 0)
            pltpu.sync_copy(denb,
                            dens[h].at[pl.ds(cid * NP + sid * NPS, NPS)])
            plsc.subcore_barrier()

    return body


def _edge(hlist, atts, attd, src, dst, H):
    f = pl.kernel(
        _make_edge_body(H),
        out_type=tuple(
            [jax.ShapeDtypeStruct((NC, NP, 128), _F32) for _ in range(H)]
            + [jax.ShapeDtypeStruct((NC * NP,), _F32) for _ in range(H)]
            + [jax.ShapeDtypeStruct((H, 2, NP), _F32),
               jax.ShapeDtypeStruct((NC, NS, NP), _F32),
               jax.ShapeDtypeStruct((NC, NS, 16), _F32)]),
        mesh=_mesh(),
        compiler_params=_params(),
        scratch_types=[
            pltpu.VMEM((CHUNK,), _I32),
            pltpu.VMEM((CHUNK,), _I32),
            pltpu.VMEM((CHUNK, 128), _F32),
            pltpu.VMEM((CHUNK,), _F32),
            pltpu.VMEM((NP,), _F32),
            pltpu.VMEM((NP,), _F32),
            pltpu.VMEM((NP,), _F32),
            pltpu.VMEM((16, 128), _F32),
            pltpu.VMEM((H * 128,), _F32),
            pltpu.VMEM((H * 128,), _F32),
            pltpu.VMEM((NPS,), _F32),
            pltpu.VMEM((NPS,), _F32),
            pltpu.VMEM((NPS,), _F32),
            pltpu.VMEM((NPS,), _F32),
            pltpu.VMEM((NS, 16), _F32),
            pltpu.VMEM((16,), _F32),
            pltpu.VMEM_SHARED((NP, 128), _F32),
            pltpu.SemaphoreType.DMA,
        ],
    )
    outs = f(*hlist, atts, attd, src, dst)
    return outs[:H], outs[H:2 * H]


def _make_fin1_body(H):
    def body(*refs):
        nums = refs[0:H]
        dens = refs[H:2 * H]
        b1 = refs[2 * H]
        outs = refs[2 * H + 1:2 * H + 1 + H]
        nb0, nb1, db0, db1, xcb, bbuf = refs[2 * H + 1 + H:]
        cid = lax.axis_index("c")
        sid = lax.axis_index("s")
        wid = sid * NC + cid
        pltpu.sync_copy(b1, bbuf)

        def _chunk(c, _):
            n0 = wid * NPW + c * 32
            for h in range(H):
                pltpu.sync_copy(nums[h].at[0, pl.ds(n0, 32), :], nb0)
                pltpu.sync_copy(nums[h].at[1, pl.ds(n0, 32), :], nb1)
                pltpu.sync_copy(dens[h].at[pl.ds(n0, 32)], db0)
                pltpu.sync_copy(dens[h].at[pl.ds(NP + n0, 32)], db1)

                def _node(n, _, h=h):
                    dv = _splat(db0, n) + _splat(db1, n) + 1e-16
                    for r in range(8):
                        v = (nb0[n, pl.ds(r * 16, 16)]
                             + nb1[n, pl.ds(r * 16, 16)]) / dv
                        v = v + bbuf[pl.ds(h * 128 + r * 16, 16)]
                        xcb[n, pl.ds(r * 16, 16)] = jnp.maximum(v, 0.0)
                    return 0
                lax.fori_loop(0, 32, _node, 0)
                pltpu.sync_copy(xcb, outs[h].at[pl.ds(n0, 32), :])
            return 0
        lax.fori_loop(0, NPW // 32, _chunk, 0)

    return body


def _fin1(nums, dens, b1, H):
    f = pl.kernel(
        _make_fin1_body(H),
        out_type=tuple(jax.ShapeDtypeStruct((NP, 128), _F32)
                       for _ in range(H)),
        mesh=_mesh(),
        compiler_params=_params(),
        scratch_types=[
            pltpu.VMEM((32, 128), _F32),
            pltpu.VMEM((32, 128), _F32),
            pltpu.VMEM((32,), _F32),
            pltpu.VMEM((32,), _F32),
            pltpu.VMEM((32, 128), _F32),
            pltpu.VMEM((H * 128,), _F32),
        ],
    )
    outs = f(*nums, *dens, b1)
    return outs if H > 1 else (outs,)


def _fin2_body(num, den, b2, gout, nb0, nb1, db0, db1, gbuf, bbuf):
    cid = lax.axis_index("c")
    sid = lax.axis_index("s")
    wid = sid * NC + cid
    pltpu.sync_copy(b2, bbuf)

    def _chunk(c, gacc):
        n0 = wid * NPW + c * 32
        pltpu.sync_copy(num.at[0, pl.ds(n0, 32), :], nb0)
        pltpu.sync_copy(num.at[1, pl.ds(n0, 32), :], nb1)
        pltpu.sync_copy(den.at[pl.ds(n0, 32)], db0)
        pltpu.sync_copy(den.at[pl.ds(NP + n0, 32)], db1)

        def _node(n, ga):
            nid = n0 + n
            mask = jnp.full((16,), nid, _I32) < N
            dv = _splat(db0, n) + _splat(db1, n) + 1e-16
            out = []
            for r in range(8):
                v = (nb0[n, pl.ds(r * 16, 16)]
                     + nb1[n, pl.ds(r * 16, 16)]) / dv
                v = jnp.maximum(v + bbuf[pl.ds(r * 16, 16)], 0.0)
                v = jnp.where(mask, v, 0.0)
                out.append(ga[r] + v)
            return tuple(out)
        return lax.fori_loop(0, 32, _node, gacc)

    gacc = lax.fori_loop(0, NPW // 32, _chunk,
                         tuple(_zeros16() for _ in range(8)))
    for r in range(8):
        gbuf[pl.ds(r * 16, 16)] = gacc[r]
    pltpu.sync_copy(gbuf, gout.at[pl.ds(wid * 128, 128)])


def _fin2(num, den, b2):
    f = pl.kernel(
        _fin2_body,
        out_type=jax.ShapeDtypeStruct((NW * 128,), _F32),
        mesh=_mesh(),
        compiler_params=_params(),
        scratch_types=[
            pltpu.VMEM((32, 128), _F32),
            pltpu.VMEM((32, 128), _F32),
            pltpu.VMEM((32,), _F32),
            pltpu.VMEM((32,), _F32),
            pltpu.VMEM((128,), _F32),
            pltpu.VMEM((128,), _F32),
        ],
    )
    return f(num, den, b2)


def _head_body(gpart, wsa, bsa, wsb, wma, bma, wmb, wla, bla, wlb,
               qemb, wq1, bq1, wq2, bq2, b3, out,
               gp, gbuf, s1, q1, q2v, wbig, bsmall, b3v):
    cid = lax.axis_index("c")
    sid = lax.axis_index("s")
    wid = sid * NC + cid

    @pl.when(wid == 0)
    def _():
        pltpu.sync_copy(gpart, gp)
        pltpu.sync_copy(b3, b3v)
        gacc = [_zeros16() for _ in range(8)]
        for t in range(NW):
            for r in range(8):
                gacc[r] = gacc[r] + gp[pl.ds(t * 128 + r * 16, 16)]
        for r in range(8):
            gbuf[pl.ds(r * 16, 16)] = gacc[r] * (1.0 / N)

        def mlp_a(wref, bref, kdim, jblocks, srcbuf, dstbuf, do_relu):
            pltpu.sync_copy(wref, wbig.at[pl.ds(0, kdim * jblocks * 16)])
            pltpu.sync_copy(bref, bsmall.at[pl.ds(0, jblocks * 16)])
            jdim = jblocks * 16
            for jb in range(jblocks):
                def _c(cc, a, jb=jb):
                    xs = _splat(srcbuf, cc)
                    return a + xs * wbig[pl.ds(cc * jdim + jb * 16, 16)]
                acc = lax.fori_loop(0, kdim, _c, _zeros16())
                acc = acc + bsmall[pl.ds(jb * 16, 16)]
                if do_relu:
                    acc = jnp.maximum(acc, 0.0)
                dstbuf[pl.ds(jb * 16, 16)] = acc

        def dot_vec(abuf, bref, nblocks, bias_lane):
            pltpu.sync_copy(bref, bsmall.at[pl.ds(0, nblocks * 16)])
            acc = _zeros16()
            for b in range(nblocks):
                acc = acc + (abuf[pl.ds(b * 16, 16)]
                             * bsmall[pl.ds(b * 16, 16)])
            b3vals = b3v[pl.ds(0, 16)]
            return jnp.sum(acc) + jnp.sum(
                jnp.where(lax.iota(_I32, 16) == bias_lane, b3vals, 0.0))

        mlp_a(wsa, bsa, 128, 4, gbuf, s1, True)
        st = dot_vec(s1, wsb, 4, 0)
        mlp_a(wma, bma, 128, 4, gbuf, s1, True)
        se = dot_vec(s1, wmb, 4, 1)
        mlp_a(wla, bla, 128, 4, gbuf, s1, True)
        lo = dot_vec(s1, wlb, 4, 2)

        pltpu.sync_copy(qemb, gp.at[pl.ds(0, 256)])
        mlp_a(wq1, bq1, 256, 8, gp, q1, True)
        mlp_a(wq2, bq2, 128, 8, q1, q2v, False)

        sacc = _zeros16()
        for r in range(8):
            sacc = sacc + gbuf[pl.ds(r * 16, 16)] * q2v[pl.ds(r * 16, 16)]
        s = jnp.sum(sacc)
        sv = jnp.full((16,), s, _F32)
        attn = jnp.exp(sv - sv)[0]
        se2 = se * attn
        tot = st + se2 + lo

        lane = lax.iota(_I32, 16)
        ov = jnp.where(lane == 0, jnp.full((16,), st), _zeros16())
        ov = jnp.where(lane == 1, jnp.full((16,), se2), ov)
        ov = jnp.where(lane == 2, jnp.full((16,), lo), ov)
        ov = jnp.where(lane == 3, jnp.full((16,), tot), ov)
        gbuf[pl.ds(0, 16)] = ov
        pltpu.sync_copy(gbuf.at[pl.ds(0, 16)], out)


def _head(gpart, wsa, bsa, wsb, wma, bma, wmb, wla, bla, wlb,
          qemb, wq1, bq1, wq2, bq2, b3):
    f = pl.kernel(
        _head_body,
        out_type=jax.ShapeDtypeStruct((16,), _F32),
        mesh=_mesh(),
        compiler_params=_params(),
        scratch_types=[
            pltpu.VMEM((NW * 128,), _F32),
            pltpu.VMEM((128,), _F32),
            pltpu.VMEM((64,), _F32),
            pltpu.VMEM((128,), _F32),
            pltpu.VMEM((128,), _F32),
            pltpu.VMEM((256 * 128,), _F32),
            pltpu.VMEM((128,), _F32),
            pltpu.VMEM((16,), _F32),
        ],
    )
    return f(gpart, wsa, bsa, wsb, wma, bma, wmb, wla, bla, wlb,
             qemb, wq1, bq1, wq2, bq2, b3)


def kernel(node_features, edge_index, question_emb, W1, att_src1, att_dst1,
           b1, W2, att_src2, att_dst2, b2, Wsa, bsa, Wsb, bsb, Wma, bma, Wmb,
           bmb, Wla, bla, Wlb, blb, Wq1, bq1, Wq2, bq2):
    xpad = jnp.concatenate(
        [node_features, jnp.zeros((NP - N, D), _F32)]).reshape(-1)
    src_p = jnp.concatenate(
        [edge_index[0], jnp.zeros((EPAD - E,), _I32)])
    dst_p = jnp.concatenate(
        [edge_index[1], jnp.zeros((EPAD - E,), _I32)])
    w1blocks = [W1[kb * 128:(kb + 1) * 128,
                   j * 128:(j + 1) * 128].reshape(-1)
                for kb in range(2) for j in range(4)]
    w2blocks = [W2[kb * 128:(kb + 1) * 128, :].reshape(-1)
                for kb in range(4)]
    a1s = att_src1.reshape(-1)
    a1d = att_dst1.reshape(-1)
    a2s = att_src2.reshape(-1)
    a2d = att_dst2.reshape(-1)
    b3 = jnp.concatenate([bsb, bmb, blb, jnp.zeros((13,), _F32)])

    h1 = _mm(xpad, w1blocks, 256, 4, 2, [[0, 1], [2, 3]])
    nums1, dens1 = _edge(list(h1), a1s, a1d, src_p, dst_p, 4)
    x2cols = _fin1(list(nums1), list(dens1), b1, 4)
    x2 = jnp.concatenate(x2cols, axis=1).reshape(-1)

    h2 = _mm(x2, w2blocks, 512, 1, 4, [[0]])
    nums2, dens2 = _edge([h2[0]], a2s, a2d, src_p, dst_p, 1)
    gpart = _fin2(nums2[0], dens2[0], b2)

    o16 = _head(gpart, Wsa.reshape(-1), bsa, Wsb.reshape(-1),
                Wma.reshape(-1), bma, Wmb.reshape(-1),
                Wla.reshape(-1), bla, Wlb.reshape(-1),
                question_emb, Wq1.reshape(-1), bq1, Wq2.reshape(-1), bq2,
                b3)
    structural = o16[0:1].reshape(1, 1)
    semantic = o16[1:2].reshape(1, 1)
    logic = o16[2:3].reshape(1, 1)
    total = o16[3:4].reshape(1, 1)
    return (structural, semantic, logic, total)


# speedup vs baseline: 5.6071x; 1.1499x over previous
import jax
import jax.numpy as jnp
from jax import lax
from jax.experimental import pallas as pl
from jax.experimental.pallas import tpu as pltpu
from jax.experimental.pallas import tpu_sc as plsc

N = 10000
E = 160000
D = 256
HID = 128

NC = 2
NS = 16
NW = NC * NS
NP = 10240
NPW = NP // NW
NPS = NP // NS
EPW = 5120
EPAD = EPW * NW
CHUNK = 80
BL = 128 * 128

_F32 = jnp.float32
_I32 = jnp.int32


def _mesh():
    return plsc.VectorSubcoreMesh(core_axis_name="c", subcore_axis_name="s")


def _params():
    return pltpu.CompilerParams(needs_layout_passes=False)


def _splat(ref, idx):
    return plsc.load_gather(ref, [jnp.full((16,), idx, _I32)])


def _zeros16():
    return jnp.zeros((16,), _F32)


def _lrelu(v):
    return jnp.where(v >= 0, v, 0.2 * v)


def _make_mm_body(K, JC, KB, jgroups):
    nin = 1 + KB * JC

    def body(*refs):
        x_hbm = refs[0]
        wrefs = refs[1:nin]
        outs = refs[nin:nin + JC]
        xbuf, wstage, houtA, houtB = refs[nin + JC:nin + JC + 4]
        houts = [houtA, houtB]
        cid = lax.axis_index("c")
        sid = lax.axis_index("s")
        wid = sid * NC + cid

        for gj in jgroups:
            for jj, j in enumerate(gj):
                for kb in range(KB):
                    pltpu.sync_copy(
                        wrefs[kb * JC + j],
                        wstage.at[pl.ds((jj * KB + kb) * BL, BL)])

            def _chunk(c, _):
                nbase = wid * NPW + c * 32
                pltpu.sync_copy(x_hbm.at[pl.ds(nbase * K, 32 * K)], xbuf)

                def _node(p, _):
                    n0 = 2 * p
                    n1 = 2 * p + 1
                    for jj, j in enumerate(gj):
                        accs = tuple(_zeros16() for _ in range(16))
                        for kb in range(KB):
                            woff = (jj * KB + kb) * BL

                            def _k(k, a, kb=kb, woff=woff):
                                xs0 = _splat(xbuf, n0 * K + kb * 128 + k)
                                xs1 = _splat(xbuf, n1 * K + kb * 128 + k)
                                wr = [wstage[pl.ds(
                                    woff + k * 128 + r * 16, 16)]
                                    for r in range(8)]
                                return (tuple(a[r] + xs0 * wr[r]
                                              for r in range(8))
                                        + tuple(a[8 + r] + xs1 * wr[r]
                                                for r in range(8)))
                            accs = lax.fori_loop(0, 128, _k, accs)
                        for r in range(8):
                            houts[jj][n0, pl.ds(r * 16, 16)] = accs[r]
                            houts[jj][n1, pl.ds(r * 16, 16)] = accs[8 + r]
                    return 0
                lax.fori_loop(0, 16, _node, 0)
                for jj, j in enumerate(gj):
                    pltpu.sync_copy(houts[jj],
                                    outs[j].at[pl.ds(nbase, 32), :])
                return 0
            lax.fori_loop(0, NPW // 32, _chunk, 0)

    return body


def _mm(x_flat, wblocks, K, JC, KB, jgroups):
    f = pl.kernel(
        _make_mm_body(K, JC, KB, jgroups),
        out_type=tuple(jax.ShapeDtypeStruct((NP, 128), _F32)
                       for _ in range(JC)),
        mesh=_mesh(),
        compiler_params=_params(),
        scratch_types=[
            pltpu.VMEM((32 * K,), _F32),
            pltpu.VMEM((2 * 2 * BL,), _F32),
            pltpu.VMEM((32, 128), _F32),
            pltpu.VMEM((32, 128), _F32),
        ],
    )
    outs = f(x_flat, *wblocks)
    if JC == 1 and not isinstance(outs, (tuple, list)):
        outs = (outs,)
    return outs


def _make_edge_body(H):
    def body(*refs):
        hrefs = refs[0:H]
        atts_h, attd_h, src_h, dst_h = refs[H:H + 4]
        nums = refs[H + 4:H + 4 + H]
        dens = refs[H + 4 + H:H + 4 + 2 * H]
        atab, dgrid, mxg = refs[H + 4 + 2 * H:H + 7 + 2 * H]
        (idxv, dstv, rows, exv, asrc_v, adst_v, denom_v, hbuf,
         attsv, attdv, aslocal, adlocal, dgr, denb, mgv, maxb,
         acc, sem) = refs[H + 7 + 2 * H:]
        cid = lax.axis_index("c")
        sid = lax.axis_index("s")
        wid = sid * NC + cid

        pltpu.sync_copy(atts_h, attsv)
        pltpu.sync_copy(attd_h, attdv)

        pmax = jnp.full((16,), -1e30, _F32)
        for h in range(H):
            def _pch(c, carry, h=h):
                mxs, mxd = carry
                n0 = sid * NPS + c * 16
                pltpu.sync_copy(hrefs[h].at[pl.ds(n0, 16), :], hbuf)
                accs = _zeros16()
                accd = _zeros16()

                def _cc(ccol, car, h=h):
                    a_s, a_d = car
                    ridx = lax.iota(_I32, 16)
                    hv = plsc.load_gather(
                        hbuf, [ridx, jnp.full((16,), ccol, _I32)])
                    ws = _splat(attsv, h * 128 + ccol)
                    wd = _splat(attdv, h * 128 + ccol)
                    return (a_s + hv * ws, a_d + hv * wd)
                accs, accd = lax.fori_loop(0, 128, _cc, (accs, accd))
                aslocal[pl.ds(c * 16, 16)] = accs
                adlocal[pl.ds(c * 16, 16)] = accd
                mxs = jnp.maximum(mxs, accs)
                mxd = jnp.maximum(mxd, accd)
                return (mxs, mxd)
            mxs, mxd = lax.fori_loop(
                0, NPS // 16, _pch,
                (jnp.full((16,), -1e30, _F32), jnp.full((16,), -1e30, _F32)))
            pltpu.sync_copy(aslocal, atab.at[h, 0, pl.ds(sid * NPS, NPS)])
            pltpu.sync_copy(adlocal, atab.at[h, 1, pl.ds(sid * NPS, NPS)])
            ms = jnp.max(mxs)
            md = jnp.max(mxd)
            lane = lax.iota(_I32, 16)
            pmax = jnp.where(lane == 2 * h, jnp.full((16,), ms), pmax)
            pmax = jnp.where(lane == 2 * h + 1, jnp.full((16,), md), pmax)
        maxb[pl.ds(0, 16)] = pmax
        pltpu.sync_copy(maxb, mxg.at[cid, sid])
        plsc.subcore_barrier()

        pltpu.sync_copy(mxg.at[cid], mgv)
        macc = mgv[0]
        for r in range(1, NS):
            macc = jnp.maximum(macc, mgv[r])
        maxb[pl.ds(0, 16)] = macc

        for h in range(H):
            pltpu.sync_copy(atab.at[h, 0], asrc_v)
            pltpu.sync_copy(atab.at[h, 1], adst_v)
            gv = _lrelu(_splat(maxb, 2 * h) + _splat(maxb, 2 * h + 1))

            def _zd(i, _):
                denom_v[pl.ds(i * 16, 16)] = _zeros16()
                return 0
            lax.fori_loop(0, NP // 16, _zd, 0)

            def _zr(i, _):
                rr = i // 8
                cc = (i % 8) * 16
                rows[rr, pl.ds(cc, 16)] = _zeros16()
                return 0
            lax.fori_loop(0, CHUNK * 8, _zr, 0)
            for k in range(NPS // CHUNK):
                pltpu.sync_copy(
                    rows, acc.at[pl.ds(sid * NPS + k * CHUNK, CHUNK), :])
            plsc.subcore_barrier()

            def _chunk(c, _, h=h, gv=gv):
                off = wid * EPW + c * CHUNK
                pltpu.sync_copy(src_h.at[pl.ds(off, CHUNK)], idxv)
                pltpu.sync_copy(dst_h.at[pl.ds(off, CHUNK)], dstv)
                pltpu.async_copy(hrefs[h].at[idxv], rows, sem).wait()

                def _ex(j, _):
                    s16 = idxv[pl.ds(j * 16, 16)]
                    d16 = dstv[pl.ds(j * 16, 16)]
                    a = (plsc.load_gather(asrc_v, [s16])
                         + plsc.load_gather(adst_v, [d16]))
                    e = jnp.exp(_lrelu(a) - gv)
                    eid = off + j * 16 + lax.iota(_I32, 16)
                    e = jnp.where(eid < E, e, 0.0)
                    exv[pl.ds(j * 16, 16)] = e
                    plsc.addupdate_scatter(denom_v, [d16], e)
                    return 0
                lax.fori_loop(0, CHUNK // 16, _ex, 0)

                def _scale(t, _):
                    f = _splat(exv, t)
                    for r in range(8):
                        rows[t, pl.ds(r * 16, 16)] = (
                            rows[t, pl.ds(r * 16, 16)] * f)
                    return 0
                lax.fori_loop(0, CHUNK, _scale, 0)

                pltpu.async_copy(rows, acc.at[dstv], sem, add=True).wait()
                return 0
            lax.fori_loop(0, EPW // CHUNK, _chunk, 0)
            plsc.subcore_barrier()

            pltpu.sync_copy(
                acc.at[pl.ds(sid * NPS, NPS), :],
                nums[h].at[cid, pl.ds(sid * NPS, NPS), :])
            pltpu.sync_copy(denom_v, dgrid.at[cid, sid])
            plsc.subcore_barrier()

            def _zb(i, _):
                denb[pl.ds(i * 16, 16)] = _zeros16()
                return 0
            lax.fori_loop(0, NPS // 16, _zb, 0)
            for r in range(NS):
                pltpu.sync_copy(dgrid.at[cid, r, pl.ds(sid * NPS, NPS)],
                                dgr)

                def _dred(i, _):
                    denb[pl.ds(i * 16, 16)] = (
                        denb[pl.ds(i * 16, 16)] + dgr[pl.ds(i * 16, 16)])
                    return 0
                lax.fori_loop(0, NPS // 16, _dred, 0)
            pltpu.sync_copy(denb,
                            dens[h].at[pl.ds(cid * NP + sid * NPS, NPS)])
            plsc.subcore_barrier()

    return body


def _edge(hlist, atts, attd, src, dst, H):
    f = pl.kernel(
        _make_edge_body(H),
        out_type=tuple(
            [jax.ShapeDtypeStruct((NC, NP, 128), _F32) for _ in range(H)]
            + [jax.ShapeDtypeStruct((NC * NP,), _F32) for _ in range(H)]
            + [jax.ShapeDtypeStruct((H, 2, NP), _F32),
               jax.ShapeDtypeStruct((NC, NS, NP), _F32),
               jax.ShapeDtypeStruct((NC, NS, 16), _F32)]),
        mesh=_mesh(),
        compiler_params=_params(),
        scratch_types=[
            pltpu.VMEM((CHUNK,), _I32),
            pltpu.VMEM((CHUNK,), _I32),
            pltpu.VMEM((CHUNK, 128), _F32),
            pltpu.VMEM((CHUNK,), _F32),
            pltpu.VMEM((NP,), _F32),
            pltpu.VMEM((NP,), _F32),
            pltpu.VMEM((NP,), _F32),
            pltpu.VMEM((16, 128), _F32),
            pltpu.VMEM((H * 128,), _F32),
            pltpu.VMEM((H * 128,), _F32),
            pltpu.VMEM((NPS,), _F32),
            pltpu.VMEM((NPS,), _F32),
            pltpu.VMEM((NPS,), _F32),
            pltpu.VMEM((NPS,), _F32),
            pltpu.VMEM((NS, 16), _F32),
            pltpu.VMEM((16,), _F32),
            pltpu.VMEM_SHARED((NP, 128), _F32),
            pltpu.SemaphoreType.DMA,
        ],
    )
    outs = f(*hlist, atts, attd, src, dst)
    return outs[:H], outs[H:2 * H]


def _make_fin1_body(H):
    def body(*refs):
        nums = refs[0:H]
        dens = refs[H:2 * H]
        b1 = refs[2 * H]
        outs = refs[2 * H + 1:2 * H + 1 + H]
        nb0, nb1, db0, db1, xcb, bbuf = refs[2 * H + 1 + H:]
        cid = lax.axis_index("c")
        sid = lax.axis_index("s")
        wid = sid * NC + cid
        pltpu.sync_copy(b1, bbuf)

        def _chunk(c, _):
            n0 = wid * NPW + c * 32
            for h in range(H):
                pltpu.sync_copy(nums[h].at[0, pl.ds(n0, 32), :], nb0)
                pltpu.sync_copy(nums[h].at[1, pl.ds(n0, 32), :], nb1)
                pltpu.sync_copy(dens[h].at[pl.ds(n0, 32)], db0)
                pltpu.sync_copy(dens[h].at[pl.ds(NP + n0, 32)], db1)

                def _node(n, _, h=h):
                    dv = _splat(db0, n) + _splat(db1, n) + 1e-16
                    for r in range(8):
                        v = (nb0[n, pl.ds(r * 16, 16)]
                             + nb1[n, pl.ds(r * 16, 16)]) / dv
                        v = v + bbuf[pl.ds(h * 128 + r * 16, 16)]
                        xcb[n, pl.ds(r * 16, 16)] = jnp.maximum(v, 0.0)
                    return 0
                lax.fori_loop(0, 32, _node, 0)
                pltpu.sync_copy(xcb, outs[h].at[pl.ds(n0, 32), :])
            return 0
        lax.fori_loop(0, NPW // 32, _chunk, 0)

    return body


def _fin1(nums, dens, b1, H):
    f = pl.kernel(
        _make_fin1_body(H),
        out_type=tuple(jax.ShapeDtypeStruct((NP, 128), _F32)
                       for _ in range(H)),
        mesh=_mesh(),
        compiler_params=_params(),
        scratch_types=[
            pltpu.VMEM((32, 128), _F32),
            pltpu.VMEM((32, 128), _F32),
            pltpu.VMEM((32,), _F32),
            pltpu.VMEM((32,), _F32),
            pltpu.VMEM((32, 128), _F32),
            pltpu.VMEM((H * 128,), _F32),
        ],
    )
    outs = f(*nums, *dens, b1)
    return outs if H > 1 else (outs,)


def _fin2_body(num, den, b2, gout, nb0, nb1, db0, db1, gbuf, bbuf):
    cid = lax.axis_index("c")
    sid = lax.axis_index("s")
    wid = sid * NC + cid
    pltpu.sync_copy(b2, bbuf)

    def _chunk(c, gacc):
        n0 = wid * NPW + c * 32
        pltpu.sync_copy(num.at[0, pl.ds(n0, 32), :], nb0)
        pltpu.sync_copy(num.at[1, pl.ds(n0, 32), :], nb1)
        pltpu.sync_copy(den.at[pl.ds(n0, 32)], db0)
        pltpu.sync_copy(den.at[pl.ds(NP + n0, 32)], db1)

        def _node(n, ga):
            nid = n0 + n
            mask = jnp.full((16,), nid, _I32) < N
            dv = _splat(db0, n) + _splat(db1, n) + 1e-16
            out = []
            for r in range(8):
                v = (nb0[n, pl.ds(r * 16, 16)]
                     + nb1[n, pl.ds(r * 16, 16)]) / dv
                v = jnp.maximum(v + bbuf[pl.ds(r * 16, 16)], 0.0)
                v = jnp.where(mask, v, 0.0)
                out.append(ga[r] + v)
            return tuple(out)
        return lax.fori_loop(0, 32, _node, gacc)

    gacc = lax.fori_loop(0, NPW // 32, _chunk,
                         tuple(_zeros16() for _ in range(8)))
    for r in range(8):
        gbuf[pl.ds(r * 16, 16)] = gacc[r]
    pltpu.sync_copy(gbuf, gout.at[pl.ds(wid * 128, 128)])


def _fin2(num, den, b2):
    f = pl.kernel(
        _fin2_body,
        out_type=jax.ShapeDtypeStruct((NW * 128,), _F32),
        mesh=_mesh(),
        compiler_params=_params(),
        scratch_types=[
            pltpu.VMEM((32, 128), _F32),
            pltpu.VMEM((32, 128), _F32),
            pltpu.VMEM((32,), _F32),
            pltpu.VMEM((32,), _F32),
            pltpu.VMEM((128,), _F32),
            pltpu.VMEM((128,), _F32),
        ],
    )
    return f(num, den, b2)


def _head_body(gpart, wsa, bsa, wsb, wma, bma, wmb, wla, bla, wlb,
               qemb, wq1, bq1, wq2, bq2, b3, out,
               gp, gbuf, s1, q1, q2v, wbig, bsmall, b3v):
    cid = lax.axis_index("c")
    sid = lax.axis_index("s")
    wid = sid * NC + cid

    @pl.when(wid == 0)
    def _():
        pltpu.sync_copy(gpart, gp)
        pltpu.sync_copy(b3, b3v)
        gacc = [_zeros16() for _ in range(8)]
        for t in range(NW):
            for r in range(8):
                gacc[r] = gacc[r] + gp[pl.ds(t * 128 + r * 16, 16)]
        for r in range(8):
            gbuf[pl.ds(r * 16, 16)] = gacc[r] * (1.0 / N)

        def mlp_a(wref, bref, kdim, jblocks, srcbuf, dstbuf, do_relu):
            pltpu.sync_copy(wref, wbig.at[pl.ds(0, kdim * jblocks * 16)])
            pltpu.sync_copy(bref, bsmall.at[pl.ds(0, jblocks * 16)])
            jdim = jblocks * 16
            for jb in range(jblocks):
                def _c(cc, a, jb=jb):
                    xs = _splat(srcbuf, cc)
                    return a + xs * wbig[pl.ds(cc * jdim + jb * 16, 16)]
                acc = lax.fori_loop(0, kdim, _c, _zeros16())
                acc = acc + bsmall[pl.ds(jb * 16, 16)]
                if do_relu:
                    acc = jnp.maximum(acc, 0.0)
                dstbuf[pl.ds(jb * 16, 16)] = acc

        def dot_vec(abuf, bref, nblocks, bias_lane):
            pltpu.sync_copy(bref, bsmall.at[pl.ds(0, nblocks * 16)])
            acc = _zeros16()
            for b in range(nblocks):
                acc = acc + (abuf[pl.ds(b * 16, 16)]
                             * bsmall[pl.ds(b * 16, 16)])
            b3vals = b3v[pl.ds(0, 16)]
            return jnp.sum(acc) + jnp.sum(
                jnp.where(lax.iota(_I32, 16) == bias_lane, b3vals, 0.0))

        mlp_a(wsa, bsa, 128, 4, gbuf, s1, True)
        st = dot_vec(s1, wsb, 4, 0)
        mlp_a(wma, bma, 128, 4, gbuf, s1, True)
        se = dot_vec(s1, wmb, 4, 1)
        mlp_a(wla, bla, 128, 4, gbuf, s1, True)
        lo = dot_vec(s1, wlb, 4, 2)

        pltpu.sync_copy(qemb, gp.at[pl.ds(0, 256)])
        mlp_a(wq1, bq1, 256, 8, gp, q1, True)
        mlp_a(wq2, bq2, 128, 8, q1, q2v, False)

        sacc = _zeros16()
        for r in range(8):
            sacc = sacc + gbuf[pl.ds(r * 16, 16)] * q2v[pl.ds(r * 16, 16)]
        s = jnp.sum(sacc)
        sv = jnp.full((16,), s, _F32)
        attn = jnp.exp(sv - sv)[0]
        se2 = se * attn
        tot = st + se2 + lo

        lane = lax.iota(_I32, 16)
        ov = jnp.where(lane == 0, jnp.full((16,), st), _zeros16())
        ov = jnp.where(lane == 1, jnp.full((16,), se2), ov)
        ov = jnp.where(lane == 2, jnp.full((16,), lo), ov)
        ov = jnp.where(lane == 3, jnp.full((16,), tot), ov)
        gbuf[pl.ds(0, 16)] = ov
        pltpu.sync_copy(gbuf.at[pl.ds(0, 16)], out)


def _head(gpart, wsa, bsa, wsb, wma, bma, wmb, wla, bla, wlb,
          qemb, wq1, bq1, wq2, bq2, b3):
    f = pl.kernel(
        _head_body,
        out_type=jax.ShapeDtypeStruct((16,), _F32),
        mesh=_mesh(),
        compiler_params=_params(),
        scratch_types=[
            pltpu.VMEM((NW * 128,), _F32),
            pltpu.VMEM((128,), _F32),
            pltpu.VMEM((64,), _F32),
            pltpu.VMEM((128,), _F32),
            pltpu.VMEM((128,), _F32),
            pltpu.VMEM((256 * 128,), _F32),
            pltpu.VMEM((128,), _F32),
            pltpu.VMEM((16,), _F32),
        ],
    )
    return f(gpart, wsa, bsa, wsb, wma, bma, wmb, wla, bla, wlb,
             qemb, wq1, bq1, wq2, bq2, b3)


def kernel(node_features, edge_index, question_emb, W1, att_src1, att_dst1,
           b1, W2, att_src2, att_dst2, b2, Wsa, bsa, Wsb, bsb, Wma, bma, Wmb,
           bmb, Wla, bla, Wlb, blb, Wq1, bq1, Wq2, bq2):
    xpad = jnp.concatenate(
        [node_features, jnp.zeros((NP - N, D), _F32)]).reshape(-1)
    src_p = jnp.concatenate(
        [edge_index[0], jnp.zeros((EPAD - E,), _I32)])
    dst_p = jnp.concatenate(
        [edge_index[1], jnp.zeros((EPAD - E,), _I32)])
    w1blocks = [W1[kb * 128:(kb + 1) * 128,
                   j * 128:(j + 1) * 128].reshape(-1)
                for kb in range(2) for j in range(4)]
    w2blocks = [W2[kb * 128:(kb + 1) * 128, :].reshape(-1)
                for kb in range(4)]
    a1s = att_src1.reshape(-1)
    a1d = att_dst1.reshape(-1)
    a2s = att_src2.reshape(-1)
    a2d = att_dst2.reshape(-1)
    b3 = jnp.concatenate([bsb, bmb, blb, jnp.zeros((13,), _F32)])

    h1 = _mm(xpad, w1blocks, 256, 4, 2, [[0, 1], [2, 3]])
    nums1, dens1 = _edge(list(h1), a1s, a1d, src_p, dst_p, 4)
    x2cols = _fin1(list(nums1), list(dens1), b1, 4)
    x2 = jnp.concatenate(x2cols, axis=1).reshape(-1)

    h2 = _mm(x2, w2blocks, 512, 1, 4, [[0]])
    nums2, dens2 = _edge([h2[0]], a2s, a2d, src_p, dst_p, 1)
    gpart = _fin2(nums2[0], dens2[0], b2)

    o16 = _head(gpart, Wsa.reshape(-1), bsa, Wsb.reshape(-1),
                Wma.reshape(-1), bma, Wmb.reshape(-1),
                Wla.reshape(-1), bla, Wlb.reshape(-1),
                question_emb, Wq1.reshape(-1), bq1, Wq2.reshape(-1), bq2,
                b3)
    structural = o16[0:1].reshape(1, 1)
    semantic = o16[1:2].reshape(1, 1)
    logic = o16[2:3].reshape(1, 1)
    total = o16[3:4].reshape(1, 1)
    return (structural, semantic, logic, total)


# speedup vs baseline: 5.8311x; 1.0399x over previous
import jax
import jax.numpy as jnp
from jax import lax
from jax.experimental import pallas as pl
from jax.experimental.pallas import tpu as pltpu
from jax.experimental.pallas import tpu_sc as plsc

N = 10000
E = 160000
D = 256
HID = 128

NC = 2
NS = 16
NW = NC * NS
NP = 10240
NPW = NP // NW
NPS = NP // NS
EPW = 5120
EPAD = EPW * NW
CHUNK = 80
BL = 128 * 128

_F32 = jnp.float32
_I32 = jnp.int32


def _mesh():
    return plsc.VectorSubcoreMesh(core_axis_name="c", subcore_axis_name="s")


def _params():
    return pltpu.CompilerParams(needs_layout_passes=False)


def _splat(ref, idx):
    return plsc.load_gather(ref, [jnp.full((16,), idx, _I32)])


def _zeros16():
    return jnp.zeros((16,), _F32)


def _lrelu(v):
    return jnp.where(v >= 0, v, 0.2 * v)


def _make_mm_body(K, JC, KB, jgroups):
    nin = 1 + KB * JC

    def body(*refs):
        x_hbm = refs[0]
        wrefs = refs[1:nin]
        outs = refs[nin:nin + JC]
        xbuf, wstage, houtA, houtB = refs[nin + JC:nin + JC + 4]
        houts = [houtA, houtB]
        cid = lax.axis_index("c")
        sid = lax.axis_index("s")
        wid = sid * NC + cid

        for gj in jgroups:
            for jj, j in enumerate(gj):
                for kb in range(KB):
                    pltpu.sync_copy(
                        wrefs[kb * JC + j],
                        wstage.at[pl.ds((jj * KB + kb) * BL, BL)])

            def _chunk(c, _):
                nbase = wid * NPW + c * 32
                pltpu.sync_copy(x_hbm.at[pl.ds(nbase * K, 32 * K)], xbuf)

                def _node(p, _):
                    nn = [4 * p + i for i in range(4)]
                    for jj, j in enumerate(gj):
                        accs = tuple(_zeros16() for _ in range(32))
                        for kb in range(KB):
                            woff = (jj * KB + kb) * BL

                            def _k(k, a, kb=kb, woff=woff):
                                xs = [_splat(xbuf, n * K + kb * 128 + k)
                                      for n in nn]
                                wr = [wstage[pl.ds(
                                    woff + k * 128 + r * 16, 16)]
                                    for r in range(8)]
                                out = []
                                for i in range(4):
                                    out.extend(a[8 * i + r] + xs[i] * wr[r]
                                               for r in range(8))
                                return tuple(out)
                            accs = lax.fori_loop(0, 128, _k, accs)
                        for i in range(4):
                            for r in range(8):
                                houts[jj][nn[i], pl.ds(r * 16, 16)] = (
                                    accs[8 * i + r])
                    return 0
                lax.fori_loop(0, 8, _node, 0)
                for jj, j in enumerate(gj):
                    pltpu.sync_copy(houts[jj],
                                    outs[j].at[pl.ds(nbase, 32), :])
                return 0
            lax.fori_loop(0, NPW // 32, _chunk, 0)

    return body


def _mm(x_flat, wblocks, K, JC, KB, jgroups):
    f = pl.kernel(
        _make_mm_body(K, JC, KB, jgroups),
        out_type=tuple(jax.ShapeDtypeStruct((NP, 128), _F32)
                       for _ in range(JC)),
        mesh=_mesh(),
        compiler_params=_params(),
        scratch_types=[
            pltpu.VMEM((32 * K,), _F32),
            pltpu.VMEM((2 * 2 * BL,), _F32),
            pltpu.VMEM((32, 128), _F32),
            pltpu.VMEM((32, 128), _F32),
        ],
    )
    outs = f(x_flat, *wblocks)
    if JC == 1 and not isinstance(outs, (tuple, list)):
        outs = (outs,)
    return outs


def _make_edge_body(H):
    def body(*refs):
        hrefs = refs[0:H]
        atts_h, attd_h, src_h, dst_h = refs[H:H + 4]
        nums = refs[H + 4:H + 4 + H]
        dens = refs[H + 4 + H:H + 4 + 2 * H]
        atab, dgrid, mxg = refs[H + 4 + 2 * H:H + 7 + 2 * H]
        (idxv, dstv, rows, exv, asrc_v, adst_v, denom_v, hbuf,
         attsv, attdv, aslocal, adlocal, dgr, denb, mgv, maxb,
         acc, sem) = refs[H + 7 + 2 * H:]
        cid = lax.axis_index("c")
        sid = lax.axis_index("s")
        wid = sid * NC + cid

        pltpu.sync_copy(atts_h, attsv)
        pltpu.sync_copy(attd_h, attdv)

        pmax = jnp.full((16,), -1e30, _F32)
        for h in range(H):
            def _pch(c, carry, h=h):
                mxs, mxd = carry
                n0 = sid * NPS + c * 16
                pltpu.sync_copy(hrefs[h].at[pl.ds(n0, 16), :], hbuf)
                accs = _zeros16()
                accd = _zeros16()

                def _cc(ccol, car, h=h):
                    a_s, a_d = car
                    ridx = lax.iota(_I32, 16)
                    hv = plsc.load_gather(
                        hbuf, [ridx, jnp.full((16,), ccol, _I32)])
                    ws = _splat(attsv, h * 128 + ccol)
                    wd = _splat(attdv, h * 128 + ccol)
                    return (a_s + hv * ws, a_d + hv * wd)
                accs, accd = lax.fori_loop(0, 128, _cc, (accs, accd))
                aslocal[pl.ds(c * 16, 16)] = accs
                adlocal[pl.ds(c * 16, 16)] = accd
                mxs = jnp.maximum(mxs, accs)
                mxd = jnp.maximum(mxd, accd)
                return (mxs, mxd)
            mxs, mxd = lax.fori_loop(
                0, NPS // 16, _pch,
                (jnp.full((16,), -1e30, _F32), jnp.full((16,), -1e30, _F32)))
            pltpu.sync_copy(aslocal, atab.at[h, 0, pl.ds(sid * NPS, NPS)])
            pltpu.sync_copy(adlocal, atab.at[h, 1, pl.ds(sid * NPS, NPS)])
            ms = jnp.max(mxs)
            md = jnp.max(mxd)
            lane = lax.iota(_I32, 16)
            pmax = jnp.where(lane == 2 * h, jnp.full((16,), ms), pmax)
            pmax = jnp.where(lane == 2 * h + 1, jnp.full((16,), md), pmax)
        maxb[pl.ds(0, 16)] = pmax
        pltpu.sync_copy(maxb, mxg.at[cid, sid])
        plsc.subcore_barrier()

        pltpu.sync_copy(mxg.at[cid], mgv)
        macc = mgv[0]
        for r in range(1, NS):
            macc = jnp.maximum(macc, mgv[r])
        maxb[pl.ds(0, 16)] = macc

        for h in range(H):
            pltpu.sync_copy(atab.at[h, 0], asrc_v)
            pltpu.sync_copy(atab.at[h, 1], adst_v)
            gv = _lrelu(_splat(maxb, 2 * h) + _splat(maxb, 2 * h + 1))

            def _zd(i, _):
                denom_v[pl.ds(i * 16, 16)] = _zeros16()
                return 0
            lax.fori_loop(0, NP // 16, _zd, 0)

            def _zr(i, _):
                rr = i // 8
                cc = (i % 8) * 16
                rows[rr, pl.ds(cc, 16)] = _zeros16()
                return 0
            lax.fori_loop(0, CHUNK * 8, _zr, 0)
            for k in range(NPS // CHUNK):
                pltpu.sync_copy(
                    rows, acc.at[pl.ds(sid * NPS + k * CHUNK, CHUNK), :])
            plsc.subcore_barrier()

            def _chunk(c, _, h=h, gv=gv):
                off = wid * EPW + c * CHUNK
                pltpu.sync_copy(src_h.at[pl.ds(off, CHUNK)], idxv)
                pltpu.sync_copy(dst_h.at[pl.ds(off, CHUNK)], dstv)
                pltpu.async_copy(hrefs[h].at[idxv], rows, sem).wait()

                def _ex(j, _):
                    s16 = idxv[pl.ds(j * 16, 16)]
                    d16 = dstv[pl.ds(j * 16, 16)]
                    a = (plsc.load_gather(asrc_v, [s16])
                         + plsc.load_gather(adst_v, [d16]))
                    e = jnp.exp(_lrelu(a) - gv)
                    eid = off + j * 16 + lax.iota(_I32, 16)
                    e = jnp.where(eid < E, e, 0.0)
                    exv[pl.ds(j * 16, 16)] = e
                    plsc.addupdate_scatter(denom_v, [d16], e)
                    return 0
                lax.fori_loop(0, CHUNK // 16, _ex, 0)

                def _scale(t, _):
                    f = _splat(exv, t)
                    for r in range(8):
                        rows[t, pl.ds(r * 16, 16)] = (
                            rows[t, pl.ds(r * 16, 16)] * f)
                    return 0
                lax.fori_loop(0, CHUNK, _scale, 0)

                pltpu.async_copy(rows, acc.at[dstv], sem, add=True).wait()
                return 0
            lax.fori_loop(0, EPW // CHUNK, _chunk, 0)
            plsc.subcore_barrier()

            pltpu.sync_copy(
                acc.at[pl.ds(sid * NPS, NPS), :],
                nums[h].at[cid, pl.ds(sid * NPS, NPS), :])
            pltpu.sync_copy(denom_v, dgrid.at[cid, sid])
            plsc.subcore_barrier()

            def _zb(i, _):
                denb[pl.ds(i * 16, 16)] = _zeros16()
                return 0
            lax.fori_loop(0, NPS // 16, _zb, 0)
            for r in range(NS):
                pltpu.sync_copy(dgrid.at[cid, r, pl.ds(sid * NPS, NPS)],
                                dgr)

                def _dred(i, _):
                    denb[pl.ds(i * 16, 16)] = (
                        denb[pl.ds(i * 16, 16)] + dgr[pl.ds(i * 16, 16)])
                    return 0
                lax.fori_loop(0, NPS // 16, _dred, 0)
            pltpu.sync_copy(denb,
                            dens[h].at[pl.ds(cid * NP + sid * NPS, NPS)])
            plsc.subcore_barrier()

    return body


def _edge(hlist, atts, attd, src, dst, H):
    f = pl.kernel(
        _make_edge_body(H),
        out_type=tuple(
            [jax.ShapeDtypeStruct((NC, NP, 128), _F32) for _ in range(H)]
            + [jax.ShapeDtypeStruct((NC * NP,), _F32) for _ in range(H)]
            + [jax.ShapeDtypeStruct((H, 2, NP), _F32),
               jax.ShapeDtypeStruct((NC, NS, NP), _F32),
               jax.ShapeDtypeStruct((NC, NS, 16), _F32)]),
        mesh=_mesh(),
        compiler_params=_params(),
        scratch_types=[
            pltpu.VMEM((CHUNK,), _I32),
            pltpu.VMEM((CHUNK,), _I32),
            pltpu.VMEM((CHUNK, 128), _F32),
            pltpu.VMEM((CHUNK,), _F32),
            pltpu.VMEM((NP,), _F32),
            pltpu.VMEM((NP,), _F32),
            pltpu.VMEM((NP,), _F32),
            pltpu.VMEM((16, 128), _F32),
            pltpu.VMEM((H * 128,), _F32),
            pltpu.VMEM((H * 128,), _F32),
            pltpu.VMEM((NPS,), _F32),
            pltpu.VMEM((NPS,), _F32),
            pltpu.VMEM((NPS,), _F32),
            pltpu.VMEM((NPS,), _F32),
            pltpu.VMEM((NS, 16), _F32),
            pltpu.VMEM((16,), _F32),
            pltpu.VMEM_SHARED((NP, 128), _F32),
            pltpu.SemaphoreType.DMA,
        ],
    )
    outs = f(*hlist, atts, attd, src, dst)
    return outs[:H], outs[H:2 * H]


def _make_fin1_body(H):
    def body(*refs):
        nums = refs[0:H]
        dens = refs[H:2 * H]
        b1 = refs[2 * H]
        outs = refs[2 * H + 1:2 * H + 1 + H]
        nb0, nb1, db0, db1, xcb, bbuf = refs[2 * H + 1 + H:]
        cid = lax.axis_index("c")
        sid = lax.axis_index("s")
        wid = sid * NC + cid
        pltpu.sync_copy(b1, bbuf)

        def _chunk(c, _):
            n0 = wid * NPW + c * 32
            for h in range(H):
                pltpu.sync_copy(nums[h].at[0, pl.ds(n0, 32), :], nb0)
                pltpu.sync_copy(nums[h].at[1, pl.ds(n0, 32), :], nb1)
                pltpu.sync_copy(dens[h].at[pl.ds(n0, 32)], db0)
                pltpu.sync_copy(dens[h].at[pl.ds(NP + n0, 32)], db1)

                def _node(n, _, h=h):
                    dv = _splat(db0, n) + _splat(db1, n) + 1e-16
                    for r in range(8):
                        v = (nb0[n, pl.ds(r * 16, 16)]
                             + nb1[n, pl.ds(r * 16, 16)]) / dv
                        v = v + bbuf[pl.ds(h * 128 + r * 16, 16)]
                        xcb[n, pl.ds(r * 16, 16)] = jnp.maximum(v, 0.0)
                    return 0
                lax.fori_loop(0, 32, _node, 0)
                pltpu.sync_copy(xcb, outs[h].at[pl.ds(n0, 32), :])
            return 0
        lax.fori_loop(0, NPW // 32, _chunk, 0)

    return body


def _fin1(nums, dens, b1, H):
    f = pl.kernel(
        _make_fin1_body(H),
        out_type=tuple(jax.ShapeDtypeStruct((NP, 128), _F32)
                       for _ in range(H)),
        mesh=_mesh(),
        compiler_params=_params(),
        scratch_types=[
            pltpu.VMEM((32, 128), _F32),
            pltpu.VMEM((32, 128), _F32),
            pltpu.VMEM((32,), _F32),
            pltpu.VMEM((32,), _F32),
            pltpu.VMEM((32, 128), _F32),
            pltpu.VMEM((H * 128,), _F32),
        ],
    )
    outs = f(*nums, *dens, b1)
    return outs if H > 1 else (outs,)


def _fin2_body(num, den, b2, gout, nb0, nb1, db0, db1, gbuf, bbuf):
    cid = lax.axis_index("c")
    sid = lax.axis_index("s")
    wid = sid * NC + cid
    pltpu.sync_copy(b2, bbuf)

    def _chunk(c, gacc):
        n0 = wid * NPW + c * 32
        pltpu.sync_copy(num.at[0, pl.ds(n0, 32), :], nb0)
        pltpu.sync_copy(num.at[1, pl.ds(n0, 32), :], nb1)
        pltpu.sync_copy(den.at[pl.ds(n0, 32)], db0)
        pltpu.sync_copy(den.at[pl.ds(NP + n0, 32)], db1)

        def _node(n, ga):
            nid = n0 + n
            mask = jnp.full((16,), nid, _I32) < N
            dv = _splat(db0, n) + _splat(db1, n) + 1e-16
            out = []
            for r in range(8):
                v = (nb0[n, pl.ds(r * 16, 16)]
                     + nb1[n, pl.ds(r * 16, 16)]) / dv
                v = jnp.maximum(v + bbuf[pl.ds(r * 16, 16)], 0.0)
                v = jnp.where(mask, v, 0.0)
                out.append(ga[r] + v)
            return tuple(out)
        return lax.fori_loop(0, 32, _node, gacc)

    gacc = lax.fori_loop(0, NPW // 32, _chunk,
                         tuple(_zeros16() for _ in range(8)))
    for r in range(8):
        gbuf[pl.ds(r * 16, 16)] = gacc[r]
    pltpu.sync_copy(gbuf, gout.at[pl.ds(wid * 128, 128)])


def _fin2(num, den, b2):
    f = pl.kernel(
        _fin2_body,
        out_type=jax.ShapeDtypeStruct((NW * 128,), _F32),
        mesh=_mesh(),
        compiler_params=_params(),
        scratch_types=[
            pltpu.VMEM((32, 128), _F32),
            pltpu.VMEM((32, 128), _F32),
            pltpu.VMEM((32,), _F32),
            pltpu.VMEM((32,), _F32),
            pltpu.VMEM((128,), _F32),
            pltpu.VMEM((128,), _F32),
        ],
    )
    return f(num, den, b2)


def _head_body(gpart, wsa, bsa, wsb, wma, bma, wmb, wla, bla, wlb,
               qemb, wq1, bq1, wq2, bq2, b3, out,
               gp, gbuf, s1, q1, q2v, wbig, bsmall, b3v):
    cid = lax.axis_index("c")
    sid = lax.axis_index("s")
    wid = sid * NC + cid

    @pl.when(wid == 0)
    def _():
        pltpu.sync_copy(gpart, gp)
        pltpu.sync_copy(b3, b3v)
        gacc = [_zeros16() for _ in range(8)]
        for t in range(NW):
            for r in range(8):
                gacc[r] = gacc[r] + gp[pl.ds(t * 128 + r * 16, 16)]
        for r in range(8):
            gbuf[pl.ds(r * 16, 16)] = gacc[r] * (1.0 / N)

        def mlp_a(wref, bref, kdim, jblocks, srcbuf, dstbuf, do_relu):
            pltpu.sync_copy(wref, wbig.at[pl.ds(0, kdim * jblocks * 16)])
            pltpu.sync_copy(bref, bsmall.at[pl.ds(0, jblocks * 16)])
            jdim = jblocks * 16
            for jb in range(jblocks):
                def _c(cc, a, jb=jb):
                    xs = _splat(srcbuf, cc)
                    return a + xs * wbig[pl.ds(cc * jdim + jb * 16, 16)]
                acc = lax.fori_loop(0, kdim, _c, _zeros16())
                acc = acc + bsmall[pl.ds(jb * 16, 16)]
                if do_relu:
                    acc = jnp.maximum(acc, 0.0)
                dstbuf[pl.ds(jb * 16, 16)] = acc

        def dot_vec(abuf, bref, nblocks, bias_lane):
            pltpu.sync_copy(bref, bsmall.at[pl.ds(0, nblocks * 16)])
            acc = _zeros16()
            for b in range(nblocks):
                acc = acc + (abuf[pl.ds(b * 16, 16)]
                             * bsmall[pl.ds(b * 16, 16)])
            b3vals = b3v[pl.ds(0, 16)]
            return jnp.sum(acc) + jnp.sum(
                jnp.where(lax.iota(_I32, 16) == bias_lane, b3vals, 0.0))

        mlp_a(wsa, bsa, 128, 4, gbuf, s1, True)
        st = dot_vec(s1, wsb, 4, 0)
        mlp_a(wma, bma, 128, 4, gbuf, s1, True)
        se = dot_vec(s1, wmb, 4, 1)
        mlp_a(wla, bla, 128, 4, gbuf, s1, True)
        lo = dot_vec(s1, wlb, 4, 2)

        pltpu.sync_copy(qemb, gp.at[pl.ds(0, 256)])
        mlp_a(wq1, bq1, 256, 8, gp, q1, True)
        mlp_a(wq2, bq2, 128, 8, q1, q2v, False)

        sacc = _zeros16()
        for r in range(8):
            sacc = sacc + gbuf[pl.ds(r * 16, 16)] * q2v[pl.ds(r * 16, 16)]
        s = jnp.sum(sacc)
        sv = jnp.full((16,), s, _F32)
        attn = jnp.exp(sv - sv)[0]
        se2 = se * attn
        tot = st + se2 + lo

        lane = lax.iota(_I32, 16)
        ov = jnp.where(lane == 0, jnp.full((16,), st), _zeros16())
        ov = jnp.where(lane == 1, jnp.full((16,), se2), ov)
        ov = jnp.where(lane == 2, jnp.full((16,), lo), ov)
        ov = jnp.where(lane == 3, jnp.full((16,), tot), ov)
        gbuf[pl.ds(0, 16)] = ov
        pltpu.sync_copy(gbuf.at[pl.ds(0, 16)], out)


def _head(gpart, wsa, bsa, wsb, wma, bma, wmb, wla, bla, wlb,
          qemb, wq1, bq1, wq2, bq2, b3):
    f = pl.kernel(
        _head_body,
        out_type=jax.ShapeDtypeStruct((16,), _F32),
        mesh=_mesh(),
        compiler_params=_params(),
        scratch_types=[
            pltpu.VMEM((NW * 128,), _F32),
            pltpu.VMEM((128,), _F32),
            pltpu.VMEM((64,), _F32),
            pltpu.VMEM((128,), _F32),
            pltpu.VMEM((128,), _F32),
            pltpu.VMEM((256 * 128,), _F32),
            pltpu.VMEM((128,), _F32),
            pltpu.VMEM((16,), _F32),
        ],
    )
    return f(gpart, wsa, bsa, wsb, wma, bma, wmb, wla, bla, wlb,
             qemb, wq1, bq1, wq2, bq2, b3)


def kernel(node_features, edge_index, question_emb, W1, att_src1, att_dst1,
           b1, W2, att_src2, att_dst2, b2, Wsa, bsa, Wsb, bsb, Wma, bma, Wmb,
           bmb, Wla, bla, Wlb, blb, Wq1, bq1, Wq2, bq2):
    xpad = jnp.concatenate(
        [node_features, jnp.zeros((NP - N, D), _F32)]).reshape(-1)
    src_p = jnp.concatenate(
        [edge_index[0], jnp.zeros((EPAD - E,), _I32)])
    dst_p = jnp.concatenate(
        [edge_index[1], jnp.zeros((EPAD - E,), _I32)])
    w1blocks = [W1[kb * 128:(kb + 1) * 128,
                   j * 128:(j + 1) * 128].reshape(-1)
                for kb in range(2) for j in range(4)]
    w2blocks = [W2[kb * 128:(kb + 1) * 128, :].reshape(-1)
                for kb in range(4)]
    a1s = att_src1.reshape(-1)
    a1d = att_dst1.reshape(-1)
    a2s = att_src2.reshape(-1)
    a2d = att_dst2.reshape(-1)
    b3 = jnp.concatenate([bsb, bmb, blb, jnp.zeros((13,), _F32)])

    h1 = _mm(xpad, w1blocks, 256, 4, 2, [[0, 1], [2, 3]])
    nums1, dens1 = _edge(list(h1), a1s, a1d, src_p, dst_p, 4)
    x2cols = _fin1(list(nums1), list(dens1), b1, 4)
    x2 = jnp.concatenate(x2cols, axis=1).reshape(-1)

    h2 = _mm(x2, w2blocks, 512, 1, 4, [[0]])
    nums2, dens2 = _edge([h2[0]], a2s, a2d, src_p, dst_p, 1)
    gpart = _fin2(nums2[0], dens2[0], b2)

    o16 = _head(gpart, Wsa.reshape(-1), bsa, Wsb.reshape(-1),
                Wma.reshape(-1), bma, Wmb.reshape(-1),
                Wla.reshape(-1), bla, Wlb.reshape(-1),
                question_emb, Wq1.reshape(-1), bq1, Wq2.reshape(-1), bq2,
                b3)
    structural = o16[0:1].reshape(1, 1)
    semantic = o16[1:2].reshape(1, 1)
    logic = o16[2:3].reshape(1, 1)
    total = o16[3:4].reshape(1, 1)
    return (structural, semantic, logic, total)


# speedup vs baseline: 5.9090x; 1.0134x over previous
import jax
import jax.numpy as jnp
from jax import lax
from jax.experimental import pallas as pl
from jax.experimental.pallas import tpu as pltpu
from jax.experimental.pallas import tpu_sc as plsc

N = 10000
E = 160000
D = 256
HID = 128

NC = 2
NS = 16
NW = NC * NS
NP = 10240
NPW = NP // NW
NPS = NP // NS
EPW = 5120
EPAD = EPW * NW
CHUNK = 80
BL = 128 * 128

_F32 = jnp.float32
_I32 = jnp.int32


def _mesh():
    return plsc.VectorSubcoreMesh(core_axis_name="c", subcore_axis_name="s")


def _params():
    return pltpu.CompilerParams(needs_layout_passes=False)


def _splat(ref, idx):
    return plsc.load_gather(ref, [jnp.full((16,), idx, _I32)])


def _zeros16():
    return jnp.zeros((16,), _F32)


def _lrelu(v):
    return jnp.where(v >= 0, v, 0.2 * v)


def _make_mm_body(K, JC, KB, jgroups):
    nin = 1 + KB * JC

    def body(*refs):
        x_hbm = refs[0]
        wrefs = refs[1:nin]
        outs = refs[nin:nin + JC]
        xbuf, wstage, houtA, houtB = refs[nin + JC:nin + JC + 4]
        houts = [houtA, houtB]
        cid = lax.axis_index("c")
        sid = lax.axis_index("s")
        wid = sid * NC + cid

        for gj in jgroups:
            for jj, j in enumerate(gj):
                for kb in range(KB):
                    pltpu.sync_copy(
                        wrefs[kb * JC + j],
                        wstage.at[pl.ds((jj * KB + kb) * BL, BL)])

            def _chunk(c, _):
                nbase = wid * NPW + c * 32
                pltpu.sync_copy(x_hbm.at[pl.ds(nbase * K, 32 * K)], xbuf)

                def _node(p, _):
                    nn = [4 * p + i for i in range(4)]
                    for jj, j in enumerate(gj):
                        accs = tuple(_zeros16() for _ in range(32))
                        for kb in range(KB):
                            woff = (jj * KB + kb) * BL

                            def _k(k, a, kb=kb, woff=woff):
                                xs = [_splat(xbuf, n * K + kb * 128 + k)
                                      for n in nn]
                                wr = [wstage[pl.ds(
                                    woff + k * 128 + r * 16, 16)]
                                    for r in range(8)]
                                out = []
                                for i in range(4):
                                    out.extend(a[8 * i + r] + xs[i] * wr[r]
                                               for r in range(8))
                                return tuple(out)
                            accs = lax.fori_loop(0, 128, _k, accs)
                        for i in range(4):
                            for r in range(8):
                                houts[jj][nn[i], pl.ds(r * 16, 16)] = (
                                    accs[8 * i + r])
                    return 0
                lax.fori_loop(0, 8, _node, 0)
                for jj, j in enumerate(gj):
                    pltpu.sync_copy(houts[jj],
                                    outs[j].at[pl.ds(nbase, 32), :])
                return 0
            lax.fori_loop(0, NPW // 32, _chunk, 0)

    return body


def _mm(x_flat, wblocks, K, JC, KB, jgroups):
    f = pl.kernel(
        _make_mm_body(K, JC, KB, jgroups),
        out_type=tuple(jax.ShapeDtypeStruct((NP, 128), _F32)
                       for _ in range(JC)),
        mesh=_mesh(),
        compiler_params=_params(),
        scratch_types=[
            pltpu.VMEM((32 * K,), _F32),
            pltpu.VMEM((2 * 2 * BL,), _F32),
            pltpu.VMEM((32, 128), _F32),
            pltpu.VMEM((32, 128), _F32),
        ],
    )
    outs = f(x_flat, *wblocks)
    if JC == 1 and not isinstance(outs, (tuple, list)):
        outs = (outs,)
    return outs


def _make_edge_body(H):
    def body(*refs):
        hrefs = refs[0:H]
        atts_h, attd_h, src_h, dst_h = refs[H:H + 4]
        nums = refs[H + 4:H + 4 + H]
        dens = refs[H + 4 + H:H + 4 + 2 * H]
        atab, dgrid, mxg = refs[H + 4 + 2 * H:H + 7 + 2 * H]
        (idxv, dstv, rows, exv, asrc_v, adst_v, denom_v, hbuf,
         attsv, attdv, aslocal, adlocal, dgr, denb, mgv, maxb,
         acc, sem) = refs[H + 7 + 2 * H:]
        cid = lax.axis_index("c")
        sid = lax.axis_index("s")
        wid = sid * NC + cid

        pltpu.sync_copy(atts_h, attsv)
        pltpu.sync_copy(attd_h, attdv)

        pmax = jnp.full((16,), -1e30, _F32)
        for h in range(H):
            def _pch(c, carry, h=h):
                mxs, mxd = carry
                n0 = sid * NPS + c * 16
                pltpu.sync_copy(hrefs[h].at[pl.ds(n0, 16), :], hbuf)
                accs = _zeros16()
                accd = _zeros16()

                def _cc(ccol, car, h=h):
                    a_s, a_d = car
                    ridx = lax.iota(_I32, 16)
                    hv = plsc.load_gather(
                        hbuf, [ridx, jnp.full((16,), ccol, _I32)])
                    ws = _splat(attsv, h * 128 + ccol)
                    wd = _splat(attdv, h * 128 + ccol)
                    return (a_s + hv * ws, a_d + hv * wd)
                accs, accd = lax.fori_loop(0, 128, _cc, (accs, accd))
                aslocal[pl.ds(c * 16, 16)] = accs
                adlocal[pl.ds(c * 16, 16)] = accd
                mxs = jnp.maximum(mxs, accs)
                mxd = jnp.maximum(mxd, accd)
                return (mxs, mxd)
            mxs, mxd = lax.fori_loop(
                0, NPS // 16, _pch,
                (jnp.full((16,), -1e30, _F32), jnp.full((16,), -1e30, _F32)))
            pltpu.sync_copy(aslocal, atab.at[h, 0, pl.ds(sid * NPS, NPS)])
            pltpu.sync_copy(adlocal, atab.at[h, 1, pl.ds(sid * NPS, NPS)])
            ms = jnp.max(mxs)
            md = jnp.max(mxd)
            lane = lax.iota(_I32, 16)
            pmax = jnp.where(lane == 2 * h, jnp.full((16,), ms), pmax)
            pmax = jnp.where(lane == 2 * h + 1, jnp.full((16,), md), pmax)
        maxb[pl.ds(0, 16)] = pmax
        pltpu.sync_copy(maxb, mxg.at[cid, sid])
        plsc.subcore_barrier()

        pltpu.sync_copy(mxg.at[cid], mgv)
        macc = mgv[0]
        for r in range(1, NS):
            macc = jnp.maximum(macc, mgv[r])
        maxb[pl.ds(0, 16)] = macc

        for h in range(H):
            pltpu.sync_copy(atab.at[h, 0], asrc_v)
            pltpu.sync_copy(atab.at[h, 1], adst_v)
            gv = _lrelu(_splat(maxb, 2 * h) + _splat(maxb, 2 * h + 1))

            def _zd(i, _):
                denom_v[pl.ds(i * 16, 16)] = _zeros16()
                return 0
            lax.fori_loop(0, NP // 16, _zd, 0)

            def _zr(i, _):
                rr = i // 8
                cc = (i % 8) * 16
                rows[rr, pl.ds(cc, 16)] = _zeros16()
                return 0
            lax.fori_loop(0, CHUNK * 8, _zr, 0)
            for k in range(NPS // CHUNK):
                pltpu.sync_copy(
                    rows, acc.at[pl.ds(sid * NPS + k * CHUNK, CHUNK), :])
            plsc.subcore_barrier()

            def _chunk(c, _, h=h, gv=gv):
                off = wid * EPW + c * CHUNK
                pltpu.sync_copy(src_h.at[pl.ds(off, CHUNK)], idxv)
                pltpu.sync_copy(dst_h.at[pl.ds(off, CHUNK)], dstv)
                gat = pltpu.async_copy(hrefs[h].at[idxv], rows, sem)

                def _ex(j, _):
                    s16 = idxv[pl.ds(j * 16, 16)]
                    d16 = dstv[pl.ds(j * 16, 16)]
                    a = (plsc.load_gather(asrc_v, [s16])
                         + plsc.load_gather(adst_v, [d16]))
                    e = jnp.exp(_lrelu(a) - gv)
                    eid = off + j * 16 + lax.iota(_I32, 16)
                    e = jnp.where(eid < E, e, 0.0)
                    exv[pl.ds(j * 16, 16)] = e
                    plsc.addupdate_scatter(denom_v, [d16], e)
                    return 0
                lax.fori_loop(0, CHUNK // 16, _ex, 0)
                gat.wait()

                def _scale(t, _):
                    f = _splat(exv, t)
                    for r in range(8):
                        rows[t, pl.ds(r * 16, 16)] = (
                            rows[t, pl.ds(r * 16, 16)] * f)
                    return 0
                lax.fori_loop(0, CHUNK, _scale, 0)

                pltpu.async_copy(rows, acc.at[dstv], sem, add=True).wait()
                return 0
            lax.fori_loop(0, EPW // CHUNK, _chunk, 0)
            plsc.subcore_barrier()

            pltpu.sync_copy(
                acc.at[pl.ds(sid * NPS, NPS), :],
                nums[h].at[cid, pl.ds(sid * NPS, NPS), :])
            pltpu.sync_copy(denom_v, dgrid.at[cid, sid])
            plsc.subcore_barrier()

            def _zb(i, _):
                denb[pl.ds(i * 16, 16)] = _zeros16()
                return 0
            lax.fori_loop(0, NPS // 16, _zb, 0)
            for r in range(NS):
                pltpu.sync_copy(dgrid.at[cid, r, pl.ds(sid * NPS, NPS)],
                                dgr)

                def _dred(i, _):
                    denb[pl.ds(i * 16, 16)] = (
                        denb[pl.ds(i * 16, 16)] + dgr[pl.ds(i * 16, 16)])
                    return 0
                lax.fori_loop(0, NPS // 16, _dred, 0)
            pltpu.sync_copy(denb,
                            dens[h].at[pl.ds(cid * NP + sid * NPS, NPS)])
            plsc.subcore_barrier()

    return body


def _edge(hlist, atts, attd, src, dst, H):
    f = pl.kernel(
        _make_edge_body(H),
        out_type=tuple(
            [jax.ShapeDtypeStruct((NC, NP, 128), _F32) for _ in range(H)]
            + [jax.ShapeDtypeStruct((NC * NP,), _F32) for _ in range(H)]
            + [jax.ShapeDtypeStruct((H, 2, NP), _F32),
               jax.ShapeDtypeStruct((NC, NS, NP), _F32),
               jax.ShapeDtypeStruct((NC, NS, 16), _F32)]),
        mesh=_mesh(),
        compiler_params=_params(),
        scratch_types=[
            pltpu.VMEM((CHUNK,), _I32),
            pltpu.VMEM((CHUNK,), _I32),
            pltpu.VMEM((CHUNK, 128), _F32),
            pltpu.VMEM((CHUNK,), _F32),
            pltpu.VMEM((NP,), _F32),
            pltpu.VMEM((NP,), _F32),
            pltpu.VMEM((NP,), _F32),
            pltpu.VMEM((16, 128), _F32),
            pltpu.VMEM((H * 128,), _F32),
            pltpu.VMEM((H * 128,), _F32),
            pltpu.VMEM((NPS,), _F32),
            pltpu.VMEM((NPS,), _F32),
            pltpu.VMEM((NPS,), _F32),
            pltpu.VMEM((NPS,), _F32),
            pltpu.VMEM((NS, 16), _F32),
            pltpu.VMEM((16,), _F32),
            pltpu.VMEM_SHARED((NP, 128), _F32),
            pltpu.SemaphoreType.DMA,
        ],
    )
    outs = f(*hlist, atts, attd, src, dst)
    return outs[:H], outs[H:2 * H]


def _make_fin1_body(H):
    def body(*refs):
        nums = refs[0:H]
        dens = refs[H:2 * H]
        b1 = refs[2 * H]
        outs = refs[2 * H + 1:2 * H + 1 + H]
        nb0, nb1, db0, db1, xcb, bbuf = refs[2 * H + 1 + H:]
        cid = lax.axis_index("c")
        sid = lax.axis_index("s")
        wid = sid * NC + cid
        pltpu.sync_copy(b1, bbuf)

        def _chunk(c, _):
            n0 = wid * NPW + c * 32
            for h in range(H):
                pltpu.sync_copy(nums[h].at[0, pl.ds(n0, 32), :], nb0)
                pltpu.sync_copy(nums[h].at[1, pl.ds(n0, 32), :], nb1)
                pltpu.sync_copy(dens[h].at[pl.ds(n0, 32)], db0)
                pltpu.sync_copy(dens[h].at[pl.ds(NP + n0, 32)], db1)

                def _node(n, _, h=h):
                    dv = _splat(db0, n) + _splat(db1, n) + 1e-16
                    for r in range(8):
                        v = (nb0[n, pl.ds(r * 16, 16)]
                             + nb1[n, pl.ds(r * 16, 16)]) / dv
                        v = v + bbuf[pl.ds(h * 128 + r * 16, 16)]
                        xcb[n, pl.ds(r * 16, 16)] = jnp.maximum(v, 0.0)
                    return 0
                lax.fori_loop(0, 32, _node, 0)
                pltpu.sync_copy(xcb, outs[h].at[pl.ds(n0, 32), :])
            return 0
        lax.fori_loop(0, NPW // 32, _chunk, 0)

    return body


def _fin1(nums, dens, b1, H):
    f = pl.kernel(
        _make_fin1_body(H),
        out_type=tuple(jax.ShapeDtypeStruct((NP, 128), _F32)
                       for _ in range(H)),
        mesh=_mesh(),
        compiler_params=_params(),
        scratch_types=[
            pltpu.VMEM((32, 128), _F32),
            pltpu.VMEM((32, 128), _F32),
            pltpu.VMEM((32,), _F32),
            pltpu.VMEM((32,), _F32),
            pltpu.VMEM((32, 128), _F32),
            pltpu.VMEM((H * 128,), _F32),
        ],
    )
    outs = f(*nums, *dens, b1)
    return outs if H > 1 else (outs,)


def _fin2_body(num, den, b2, gout, nb0, nb1, db0, db1, gbuf, bbuf):
    cid = lax.axis_index("c")
    sid = lax.axis_index("s")
    wid = sid * NC + cid
    pltpu.sync_copy(b2, bbuf)

    def _chunk(c, gacc):
        n0 = wid * NPW + c * 32
        pltpu.sync_copy(num.at[0, pl.ds(n0, 32), :], nb0)
        pltpu.sync_copy(num.at[1, pl.ds(n0, 32), :], nb1)
        pltpu.sync_copy(den.at[pl.ds(n0, 32)], db0)
        pltpu.sync_copy(den.at[pl.ds(NP + n0, 32)], db1)

        def _node(n, ga):
            nid = n0 + n
            mask = jnp.full((16,), nid, _I32) < N
            dv = _splat(db0, n) + _splat(db1, n) + 1e-16
            out = []
            for r in range(8):
                v = (nb0[n, pl.ds(r * 16, 16)]
                     + nb1[n, pl.ds(r * 16, 16)]) / dv
                v = jnp.maximum(v + bbuf[pl.ds(r * 16, 16)], 0.0)
                v = jnp.where(mask, v, 0.0)
                out.append(ga[r] + v)
            return tuple(out)
        return lax.fori_loop(0, 32, _node, gacc)

    gacc = lax.fori_loop(0, NPW // 32, _chunk,
                         tuple(_zeros16() for _ in range(8)))
    for r in range(8):
        gbuf[pl.ds(r * 16, 16)] = gacc[r]
    pltpu.sync_copy(gbuf, gout.at[pl.ds(wid * 128, 128)])


def _fin2(num, den, b2):
    f = pl.kernel(
        _fin2_body,
        out_type=jax.ShapeDtypeStruct((NW * 128,), _F32),
        mesh=_mesh(),
        compiler_params=_params(),
        scratch_types=[
            pltpu.VMEM((32, 128), _F32),
            pltpu.VMEM((32, 128), _F32),
            pltpu.VMEM((32,), _F32),
            pltpu.VMEM((32,), _F32),
            pltpu.VMEM((128,), _F32),
            pltpu.VMEM((128,), _F32),
        ],
    )
    return f(num, den, b2)


def _head_body(gpart, wsa, bsa, wsb, wma, bma, wmb, wla, bla, wlb,
               qemb, wq1, bq1, wq2, bq2, b3, out,
               gp, gbuf, s1, q1, q2v, wbig, bsmall, b3v):
    cid = lax.axis_index("c")
    sid = lax.axis_index("s")
    wid = sid * NC + cid

    @pl.when(wid == 0)
    def _():
        pltpu.sync_copy(gpart, gp)
        pltpu.sync_copy(b3, b3v)
        gacc = [_zeros16() for _ in range(8)]
        for t in range(NW):
            for r in range(8):
                gacc[r] = gacc[r] + gp[pl.ds(t * 128 + r * 16, 16)]
        for r in range(8):
            gbuf[pl.ds(r * 16, 16)] = gacc[r] * (1.0 / N)

        def mlp_a(wref, bref, kdim, jblocks, srcbuf, dstbuf, do_relu):
            pltpu.sync_copy(wref, wbig.at[pl.ds(0, kdim * jblocks * 16)])
            pltpu.sync_copy(bref, bsmall.at[pl.ds(0, jblocks * 16)])
            jdim = jblocks * 16
            for jb in range(jblocks):
                def _c(cc, a, jb=jb):
                    xs = _splat(srcbuf, cc)
                    return a + xs * wbig[pl.ds(cc * jdim + jb * 16, 16)]
                acc = lax.fori_loop(0, kdim, _c, _zeros16())
                acc = acc + bsmall[pl.ds(jb * 16, 16)]
                if do_relu:
                    acc = jnp.maximum(acc, 0.0)
                dstbuf[pl.ds(jb * 16, 16)] = acc

        def dot_vec(abuf, bref, nblocks, bias_lane):
            pltpu.sync_copy(bref, bsmall.at[pl.ds(0, nblocks * 16)])
            acc = _zeros16()
            for b in range(nblocks):
                acc = acc + (abuf[pl.ds(b * 16, 16)]
                             * bsmall[pl.ds(b * 16, 16)])
            b3vals = b3v[pl.ds(0, 16)]
            return jnp.sum(acc) + jnp.sum(
                jnp.where(lax.iota(_I32, 16) == bias_lane, b3vals, 0.0))

        mlp_a(wsa, bsa, 128, 4, gbuf, s1, True)
        st = dot_vec(s1, wsb, 4, 0)
        mlp_a(wma, bma, 128, 4, gbuf, s1, True)
        se = dot_vec(s1, wmb, 4, 1)
        mlp_a(wla, bla, 128, 4, gbuf, s1, True)
        lo = dot_vec(s1, wlb, 4, 2)

        pltpu.sync_copy(qemb, gp.at[pl.ds(0, 256)])
        mlp_a(wq1, bq1, 256, 8, gp, q1, True)
        mlp_a(wq2, bq2, 128, 8, q1, q2v, False)

        sacc = _zeros16()
        for r in range(8):
            sacc = sacc + gbuf[pl.ds(r * 16, 16)] * q2v[pl.ds(r * 16, 16)]
        s = jnp.sum(sacc)
        sv = jnp.full((16,), s, _F32)
        attn = jnp.exp(sv - sv)[0]
        se2 = se * attn
        tot = st + se2 + lo

        lane = lax.iota(_I32, 16)
        ov = jnp.where(lane == 0, jnp.full((16,), st), _zeros16())
        ov = jnp.where(lane == 1, jnp.full((16,), se2), ov)
        ov = jnp.where(lane == 2, jnp.full((16,), lo), ov)
        ov = jnp.where(lane == 3, jnp.full((16,), tot), ov)
        gbuf[pl.ds(0, 16)] = ov
        pltpu.sync_copy(gbuf.at[pl.ds(0, 16)], out)


def _head(gpart, wsa, bsa, wsb, wma, bma, wmb, wla, bla, wlb,
          qemb, wq1, bq1, wq2, bq2, b3):
    f = pl.kernel(
        _head_body,
        out_type=jax.ShapeDtypeStruct((16,), _F32),
        mesh=_mesh(),
        compiler_params=_params(),
        scratch_types=[
            pltpu.VMEM((NW * 128,), _F32),
            pltpu.VMEM((128,), _F32),
            pltpu.VMEM((64,), _F32),
            pltpu.VMEM((128,), _F32),
            pltpu.VMEM((128,), _F32),
            pltpu.VMEM((256 * 128,), _F32),
            pltpu.VMEM((128,), _F32),
            pltpu.VMEM((16,), _F32),
        ],
    )
    return f(gpart, wsa, bsa, wsb, wma, bma, wmb, wla, bla, wlb,
             qemb, wq1, bq1, wq2, bq2, b3)


def kernel(node_features, edge_index, question_emb, W1, att_src1, att_dst1,
           b1, W2, att_src2, att_dst2, b2, Wsa, bsa, Wsb, bsb, Wma, bma, Wmb,
           bmb, Wla, bla, Wlb, blb, Wq1, bq1, Wq2, bq2):
    xpad = jnp.concatenate(
        [node_features, jnp.zeros((NP - N, D), _F32)]).reshape(-1)
    src_p = jnp.concatenate(
        [edge_index[0], jnp.zeros((EPAD - E,), _I32)])
    dst_p = jnp.concatenate(
        [edge_index[1], jnp.zeros((EPAD - E,), _I32)])
    w1blocks = [W1[kb * 128:(kb + 1) * 128,
                   j * 128:(j + 1) * 128].reshape(-1)
                for kb in range(2) for j in range(4)]
    w2blocks = [W2[kb * 128:(kb + 1) * 128, :].reshape(-1)
                for kb in range(4)]
    a1s = att_src1.reshape(-1)
    a1d = att_dst1.reshape(-1)
    a2s = att_src2.reshape(-1)
    a2d = att_dst2.reshape(-1)
    b3 = jnp.concatenate([bsb, bmb, blb, jnp.zeros((13,), _F32)])

    h1 = _mm(xpad, w1blocks, 256, 4, 2, [[0, 1], [2, 3]])
    nums1, dens1 = _edge(list(h1), a1s, a1d, src_p, dst_p, 4)
    x2cols = _fin1(list(nums1), list(dens1), b1, 4)
    x2 = jnp.concatenate(x2cols, axis=1).reshape(-1)

    h2 = _mm(x2, w2blocks, 512, 1, 4, [[0]])
    nums2, dens2 = _edge([h2[0]], a2s, a2d, src_p, dst_p, 1)
    gpart = _fin2(nums2[0], dens2[0], b2)

    o16 = _head(gpart, Wsa.reshape(-1), bsa, Wsb.reshape(-1),
                Wma.reshape(-1), bma, Wmb.reshape(-1),
                Wla.reshape(-1), bla, Wlb.reshape(-1),
                question_emb, Wq1.reshape(-1), bq1, Wq2.reshape(-1), bq2,
                b3)
    structural = o16[0:1].reshape(1, 1)
    semantic = o16[1:2].reshape(1, 1)
    logic = o16[2:3].reshape(1, 1)
    total = o16[3:4].reshape(1, 1)
    return (structural, semantic, logic, total)
